# Initial kernel scaffold; baseline (speedup 1.0000x reference)
import jax
import jax.numpy as jnp
from jax.experimental import pallas as pl


def kernel(inputs, W, compute_distances_if_possible):
    raise NotImplementedError("write your pallas kernel here")


# speedup vs baseline: 1.1085x; 1.1085x over previous
import jax
import jax.numpy as jnp
from jax.experimental import pallas as pl
from jax.experimental.pallas import tpu as pltpu

_NE = 1024
_ED = 256
_B = 32
_T = 1024
_N = _B * _T
_TN = 512
_GRID = _N // _TN
_CC = 0.25


def _vq_tile_kernel(x_ref, w_ref, wt_ref, w2_ref,
                    dist_ref, enc_ref, qt_ref, idx_ref, loss_ref, perp_ref,
                    hist_ref, msum_ref):
    j = pl.program_id(0)

    @pl.when(j == 0)
    def _init():
        hist_ref[...] = jnp.zeros_like(hist_ref)
        msum_ref[...] = jnp.zeros_like(msum_ref)

    x = x_ref[...]
    w = w_ref[...]
    xw = jax.lax.dot_general(x, w, (((1,), (1,)), ((), ())),
                             preferred_element_type=jnp.float32)
    x2 = jnp.sum(x * x, axis=1, keepdims=True)
    dist = (x2 + w2_ref[...]) - 2.0 * xw
    dist_ref[...] = dist

    mn = jnp.min(dist, axis=1, keepdims=True)
    iota_l = jax.lax.broadcasted_iota(jnp.int32, (_TN, _NE), 1)
    idx = jnp.min(jnp.where(dist == mn, iota_l, _NE), axis=1).astype(jnp.int32)
    idx_ref[...] = idx[:, None]

    onehot = (iota_l == idx[:, None]).astype(jnp.float32)
    enc_ref[...] = onehot

    iota_s = jax.lax.broadcasted_iota(jnp.int32, (_NE, _TN), 0)
    onehot_t = (iota_s == idx[None, :]).astype(jnp.float32)
    qt_ref[...] = jax.lax.dot_general(wt_ref[...], onehot_t,
                                      (((1,), (0,)), ((), ())),
                                      preferred_element_type=jnp.float32)

    hist_ref[...] += jnp.sum(onehot, axis=0, keepdims=True)
    msum_ref[...] += jnp.sum(mn).reshape(1, 1)

    @pl.when(j == _GRID - 1)
    def _fin():
        avg = hist_ref[...] / _N
        ent = jnp.sum(avg * jnp.log(avg + 1e-10))
        perp_ref[...] = jnp.exp(-ent).reshape(1, 1)
        m = msum_ref[...] / (_N * _ED)
        loss_ref[...] = m + _CC * m


def kernel(inputs, W, compute_distances_if_possible):
    del compute_distances_if_possible
    x_flat = jnp.transpose(inputs, (1, 2, 0)).reshape(_N, _ED)
    w2 = jnp.sum(W ** 2, axis=1)[None, :]
    wt = W.T

    dist, enc, qt, idxo, loss, perp = pl.pallas_call(
        _vq_tile_kernel,
        grid=(_GRID,),
        in_specs=[
            pl.BlockSpec((_TN, _ED), lambda j: (j, 0)),
            pl.BlockSpec((_NE, _ED), lambda j: (0, 0)),
            pl.BlockSpec((_ED, _NE), lambda j: (0, 0)),
            pl.BlockSpec((1, _NE), lambda j: (0, 0)),
        ],
        out_specs=[
            pl.BlockSpec((_TN, _NE), lambda j: (j, 0)),
            pl.BlockSpec((_TN, _NE), lambda j: (j, 0)),
            pl.BlockSpec((_ED, _TN), lambda j: (0, j)),
            pl.BlockSpec((_TN, 1), lambda j: (j, 0)),
            pl.BlockSpec((1, 1), lambda j: (0, 0)),
            pl.BlockSpec((1, 1), lambda j: (0, 0)),
        ],
        out_shape=[
            jax.ShapeDtypeStruct((_N, _NE), jnp.float32),
            jax.ShapeDtypeStruct((_N, _NE), jnp.float32),
            jax.ShapeDtypeStruct((_ED, _N), jnp.float32),
            jax.ShapeDtypeStruct((_N, 1), jnp.int32),
            jax.ShapeDtypeStruct((1, 1), jnp.float32),
            jax.ShapeDtypeStruct((1, 1), jnp.float32),
        ],
        scratch_shapes=[
            pltpu.VMEM((1, _NE), jnp.float32),
            pltpu.VMEM((1, 1), jnp.float32),
        ],
    )(x_flat, W, wt, w2)

    vq_loss = loss[0, 0]
    quantized_out = qt.reshape(_ED, _B, _T)
    perplexity = perp[0, 0]
    encodings_view = enc.reshape(_ED, _T, -1)
    distances_view = dist.reshape(_ED, _T, -1)
    return (vq_loss, quantized_out, perplexity, encodings_view,
            distances_view, idxo)


# speedup vs baseline: 1.1243x; 1.0143x over previous
import jax
import jax.numpy as jnp
from jax.experimental import pallas as pl
from jax.experimental.pallas import tpu as pltpu

_NE = 1024
_ED = 256
_B = 32
_T = 1024
_N = _B * _T
_TN = 512
_GRID = _N // _TN
_CC = 0.25


def _vq_tile_kernel(xt_ref, w_ref, wt_ref, w2_ref,
                    dist_ref, enc_ref, qt_ref, idx_ref, loss_ref, perp_ref,
                    hist_ref, msum_ref):
    j = pl.program_id(0)

    @pl.when(j == 0)
    def _init():
        hist_ref[...] = jnp.zeros_like(hist_ref)
        msum_ref[...] = jnp.zeros_like(msum_ref)

    x = xt_ref[...].T
    w = w_ref[...]
    xw = jax.lax.dot_general(x, w, (((1,), (1,)), ((), ())),
                             preferred_element_type=jnp.float32)
    x2 = jnp.sum(x * x, axis=1, keepdims=True)
    dist = (x2 + w2_ref[...]) - 2.0 * xw
    dist_ref[...] = dist

    mn = jnp.min(dist, axis=1, keepdims=True)
    iota_l = jax.lax.broadcasted_iota(jnp.int32, (_TN, _NE), 1)
    idx = jnp.min(jnp.where(dist == mn, iota_l, _NE), axis=1).astype(jnp.int32)
    idx_ref[...] = idx[None, None, :]

    onehot = (iota_l == idx[:, None]).astype(jnp.float32)
    enc_ref[...] = onehot

    iota_s = jax.lax.broadcasted_iota(jnp.int32, (_NE, _TN), 0)
    onehot_t = (iota_s == idx[None, :]).astype(jnp.float32)
    qt_ref[...] = jax.lax.dot_general(wt_ref[...], onehot_t,
                                      (((1,), (0,)), ((), ())),
                                      preferred_element_type=jnp.float32)

    hist_ref[...] += jnp.sum(onehot, axis=0, keepdims=True)
    msum_ref[...] += jnp.sum(mn).reshape(1, 1)

    @pl.when(j == _GRID - 1)
    def _fin():
        avg = hist_ref[...] / _N
        ent = jnp.sum(avg * jnp.log(avg + 1e-10))
        perp_ref[...] = jnp.exp(-ent).reshape(1, 1)
        m = msum_ref[...] / (_N * _ED)
        loss_ref[...] = m + _CC * m


def kernel(inputs, W, compute_distances_if_possible):
    del compute_distances_if_possible
    xt = inputs.reshape(_ED, _N)
    w2 = jnp.sum(W ** 2, axis=1)[None, :]
    wt = W.T

    dist, enc, qt, idxo, loss, perp = pl.pallas_call(
        _vq_tile_kernel,
        grid=(_GRID,),
        in_specs=[
            pl.BlockSpec((_ED, _TN), lambda j: (0, j)),
            pl.BlockSpec((_NE, _ED), lambda j: (0, 0)),
            pl.BlockSpec((_ED, _NE), lambda j: (0, 0)),
            pl.BlockSpec((1, _NE), lambda j: (0, 0)),
        ],
        out_specs=[
            pl.BlockSpec((_TN, _NE), lambda j: (j, 0)),
            pl.BlockSpec((_TN, _NE), lambda j: (j, 0)),
            pl.BlockSpec((_ED, _TN), lambda j: (0, j)),
            pl.BlockSpec((1, 1, _TN), lambda j: (j, 0, 0)),
            pl.BlockSpec((1, 1), lambda j: (0, 0)),
            pl.BlockSpec((1, 1), lambda j: (0, 0)),
        ],
        out_shape=[
            jax.ShapeDtypeStruct((_N, _NE), jnp.float32),
            jax.ShapeDtypeStruct((_N, _NE), jnp.float32),
            jax.ShapeDtypeStruct((_ED, _N), jnp.float32),
            jax.ShapeDtypeStruct((_GRID, 1, _TN), jnp.int32),
            jax.ShapeDtypeStruct((1, 1), jnp.float32),
            jax.ShapeDtypeStruct((1, 1), jnp.float32),
        ],
        scratch_shapes=[
            pltpu.VMEM((1, _NE), jnp.float32),
            pltpu.VMEM((1, 1), jnp.float32),
        ],
    )(xt, W, wt, w2)

    vq_loss = loss[0, 0]
    quantized_out = qt.reshape(_ED, _B, _T)
    perplexity = perp[0, 0]
    encodings_view = enc.reshape(_ED, _T, -1)
    distances_view = dist.reshape(_ED, _T, -1)
    return (vq_loss, quantized_out, perplexity, encodings_view,
            distances_view, idxo.reshape(_N, 1))


# speedup vs baseline: 2.0722x; 1.8431x over previous
import jax
import jax.numpy as jnp
from jax.experimental import pallas as pl
from jax.experimental.pallas import tpu as pltpu

_NE = 1024
_ED = 256
_B = 32
_T = 1024
_N = _B * _T
_TN = 512
_GRID = _N // _TN
_CC = 0.25


def _vq_tile_kernel(xt_ref, w_ref, wt_ref, w2_ref,
                    dist_ref, enc_ref, qt_ref, idx_ref, loss_ref, perp_ref,
                    hist_ref, msum_ref):
    j = pl.program_id(0)

    @pl.when(j == 0)
    def _init():
        hist_ref[...] = jnp.zeros_like(hist_ref)
        msum_ref[...] = jnp.zeros_like(msum_ref)

    x = xt_ref[...].T
    w = w_ref[...]
    xw = jax.lax.dot_general(x, w, (((1,), (1,)), ((), ())),
                             preferred_element_type=jnp.float32)
    x2 = jnp.sum(x * x, axis=1, keepdims=True)
    dist = (x2 + w2_ref[...]) - 2.0 * xw
    dist_ref[...] = dist.reshape(_TN // 128, 1024, 128)

    mn = jnp.min(dist, axis=1, keepdims=True)
    iota_l = jax.lax.broadcasted_iota(jnp.int32, (_TN, _NE), 1)
    idx = jnp.min(jnp.where(dist == mn, iota_l, _NE), axis=1).astype(jnp.int32)
    idx_ref[...] = idx[None, None, :]

    onehot = (iota_l == idx[:, None]).astype(jnp.float32)
    enc_ref[...] = onehot.reshape(_TN // 128, 1024, 128)

    iota_s = jax.lax.broadcasted_iota(jnp.int32, (_NE, _TN), 0)
    onehot_t = (iota_s == idx[None, :]).astype(jnp.float32)
    qt_ref[...] = jax.lax.dot_general(wt_ref[...], onehot_t,
                                      (((1,), (0,)), ((), ())),
                                      preferred_element_type=jnp.float32)

    hist_ref[...] += jnp.sum(onehot, axis=0, keepdims=True)
    msum_ref[...] += jnp.sum(mn).reshape(1, 1)

    @pl.when(j == _GRID - 1)
    def _fin():
        avg = hist_ref[...] / _N
        ent = jnp.sum(avg * jnp.log(avg + 1e-10))
        perp_ref[...] = jnp.exp(-ent).reshape(1, 1)
        m = msum_ref[...] / (_N * _ED)
        loss_ref[...] = m + _CC * m


def kernel(inputs, W, compute_distances_if_possible):
    del compute_distances_if_possible
    xt = inputs.reshape(_ED, _N)
    w2 = jnp.sum(W ** 2, axis=1)[None, :]
    wt = W.T

    dist, enc, qt, idxo, loss, perp = pl.pallas_call(
        _vq_tile_kernel,
        grid=(_GRID,),
        in_specs=[
            pl.BlockSpec((_ED, _TN), lambda j: (0, j)),
            pl.BlockSpec((_NE, _ED), lambda j: (0, 0)),
            pl.BlockSpec((_ED, _NE), lambda j: (0, 0)),
            pl.BlockSpec((1, _NE), lambda j: (0, 0)),
        ],
        out_specs=[
            pl.BlockSpec((_TN // 128, 1024, 128), lambda j: (j, 0, 0)),
            pl.BlockSpec((_TN // 128, 1024, 128), lambda j: (j, 0, 0)),
            pl.BlockSpec((_ED, _TN), lambda j: (0, j)),
            pl.BlockSpec((1, 1, _TN), lambda j: (j, 0, 0)),
            pl.BlockSpec((1, 1), lambda j: (0, 0)),
            pl.BlockSpec((1, 1), lambda j: (0, 0)),
        ],
        out_shape=[
            jax.ShapeDtypeStruct((_ED, 1024, 128), jnp.float32),
            jax.ShapeDtypeStruct((_ED, 1024, 128), jnp.float32),
            jax.ShapeDtypeStruct((_ED, _N), jnp.float32),
            jax.ShapeDtypeStruct((_GRID, 1, _TN), jnp.int32),
            jax.ShapeDtypeStruct((1, 1), jnp.float32),
            jax.ShapeDtypeStruct((1, 1), jnp.float32),
        ],
        scratch_shapes=[
            pltpu.VMEM((1, _NE), jnp.float32),
            pltpu.VMEM((1, 1), jnp.float32),
        ],
    )(xt, W, wt, w2)

    vq_loss = loss[0, 0]
    quantized_out = qt.reshape(_ED, _B, _T)
    perplexity = perp[0, 0]
    encodings_view = enc
    distances_view = dist
    return (vq_loss, quantized_out, perplexity, encodings_view,
            distances_view, idxo.reshape(_N, 1))


# speedup vs baseline: 2.7483x; 1.3262x over previous
import jax
import jax.numpy as jnp
from jax.experimental import pallas as pl
from jax.experimental.pallas import tpu as pltpu

_NE = 1024
_ED = 256
_B = 32
_T = 1024
_N = _B * _T
_CC = 0.25

_BG = 4
_TC = 8
_TN = 8 * 128


def _vq_tile_kernel(x_ref, w_ref, w2_ref,
                    dist_ref, enc_ref, qt_ref, idx_ref, loss_ref, perp_ref,
                    hist_ref, msum_ref):
    g = pl.program_id(0)
    tc = pl.program_id(1)

    @pl.when((g == 0) & (tc == 0))
    def _init():
        hist_ref[...] = jnp.zeros_like(hist_ref)
        msum_ref[...] = jnp.zeros_like(msum_ref)

    x = x_ref[...].reshape(_ED, _TN).T
    w = w_ref[...]
    xw = jax.lax.dot_general(x, w, (((1,), (1,)), ((), ())),
                             preferred_element_type=jnp.float32)
    x2 = jnp.sum(x * x, axis=1, keepdims=True)
    dist = (x2 + w2_ref[...]) - 2.0 * xw
    dist_ref[...] = dist.reshape(1, 8, 1, _NE, 128)

    mn = jnp.min(dist, axis=1, keepdims=True)
    iota_l = jax.lax.broadcasted_iota(jnp.int32, (_TN, _NE), 1)
    idx = jnp.min(jnp.where(dist == mn, iota_l, _NE), axis=1).astype(jnp.int32)
    idx_ref[...] = idx.reshape(1, 8, 1, 1, 128)

    onehot = (iota_l == idx[:, None]).astype(jnp.float32)
    enc_ref[...] = onehot.reshape(1, 8, 1, _NE, 128)

    iota_s = jax.lax.broadcasted_iota(jnp.int32, (_NE, _TN), 0)
    onehot_t = (iota_s == idx[None, :]).astype(jnp.float32)
    qt = jax.lax.dot_general(w, onehot_t, (((0,), (0,)), ((), ())),
                             preferred_element_type=jnp.float32)
    qt_ref[...] = qt.reshape(_ED, 1, 8, 128)

    hist_ref[...] += jnp.sum(onehot, axis=0, keepdims=True)
    msum_ref[...] += jnp.sum(mn).reshape(1, 1)

    @pl.when((g == _BG - 1) & (tc == _TC - 1))
    def _fin():
        avg = hist_ref[...] / _N
        ent = jnp.sum(avg * jnp.log(avg + 1e-10))
        perp_ref[...] = jnp.exp(-ent).reshape(1, 1)
        m = msum_ref[...] / (_N * _ED)
        loss_ref[...] = m + _CC * m


def kernel(inputs, W, compute_distances_if_possible):
    del compute_distances_if_possible
    x4 = inputs.reshape(_ED, _BG, 8, _T)
    w2 = jnp.sum(W ** 2, axis=1)[None, :]

    dist, enc, qt, idxo, loss, perp = pl.pallas_call(
        _vq_tile_kernel,
        grid=(_BG, _TC),
        in_specs=[
            pl.BlockSpec((_ED, 1, 8, 128), lambda g, t: (0, g, 0, t)),
            pl.BlockSpec((_NE, _ED), lambda g, t: (0, 0)),
            pl.BlockSpec((1, _NE), lambda g, t: (0, 0)),
        ],
        out_specs=[
            pl.BlockSpec((1, 8, 1, _NE, 128), lambda g, t: (g, 0, t, 0, 0)),
            pl.BlockSpec((1, 8, 1, _NE, 128), lambda g, t: (g, 0, t, 0, 0)),
            pl.BlockSpec((_ED, 1, 8, 128), lambda g, t: (0, g, 0, t)),
            pl.BlockSpec((1, 8, 1, 1, 128), lambda g, t: (g, 0, t, 0, 0)),
            pl.BlockSpec((1, 1), lambda g, t: (0, 0)),
            pl.BlockSpec((1, 1), lambda g, t: (0, 0)),
        ],
        out_shape=[
            jax.ShapeDtypeStruct((_BG, 8, _TC, _NE, 128), jnp.float32),
            jax.ShapeDtypeStruct((_BG, 8, _TC, _NE, 128), jnp.float32),
            jax.ShapeDtypeStruct((_ED, _BG, 8, _T), jnp.float32),
            jax.ShapeDtypeStruct((_BG, 8, _TC, 1, 128), jnp.int32),
            jax.ShapeDtypeStruct((1, 1), jnp.float32),
            jax.ShapeDtypeStruct((1, 1), jnp.float32),
        ],
        scratch_shapes=[
            pltpu.VMEM((1, _NE), jnp.float32),
            pltpu.VMEM((1, 1), jnp.float32),
        ],
    )(x4, W, w2)

    vq_loss = loss[0, 0]
    quantized_out = qt.reshape(_ED, _B, _T)
    perplexity = perp[0, 0]
    encodings_view = enc.reshape(_ED, _NE, 128)
    distances_view = dist.reshape(_ED, _NE, 128)
    return (vq_loss, quantized_out, perplexity, encodings_view,
            distances_view, idxo.reshape(_N, 1))


# speedup vs baseline: 2.9070x; 1.0578x over previous
import jax
import jax.numpy as jnp
from jax.experimental import pallas as pl
from jax.experimental.pallas import tpu as pltpu

_NE = 1024
_ED = 256
_B = 32
_T = 1024
_N = _B * _T
_CC = 0.25

_BG = 4
_TC = 8
_TN = 8 * 128


def _vq_tile_kernel(x_ref, w_ref, w2_ref,
                    dist_ref, enc_ref, qt_ref, idx_ref, loss_ref, perp_ref,
                    hist_ref, msum_ref):
    g = pl.program_id(0)
    tc = pl.program_id(1)

    @pl.when((g == 0) & (tc == 0))
    def _init():
        hist_ref[...] = jnp.zeros_like(hist_ref)
        msum_ref[...] = jnp.zeros_like(msum_ref)

    x = x_ref[...].reshape(_ED, _TN).T
    w = w_ref[...]
    xw2 = jax.lax.dot_general(x * (-2.0), w, (((1,), (1,)), ((), ())),
                              preferred_element_type=jnp.float32)
    x2 = jnp.sum(x * x, axis=1, keepdims=True)
    dist = (x2 + w2_ref[...]) + xw2
    dist_ref[...] = dist.reshape(1, 8, 1, _NE, 128)

    mn = jnp.min(dist, axis=1, keepdims=True)
    iota_l = jax.lax.broadcasted_iota(jnp.int32, (_TN, _NE), 1)
    idx = jnp.min(jnp.where(dist == mn, iota_l, _NE), axis=1).astype(jnp.int32)
    idx_ref[...] = idx.reshape(1, 8, 1, 1, 128)

    onehot = (iota_l == idx[:, None]).astype(jnp.float32)
    enc_ref[...] = onehot.reshape(1, 8, 1, _NE, 128)

    qt = jax.lax.dot_general(w, onehot, (((0,), (1,)), ((), ())),
                             preferred_element_type=jnp.float32)
    qt_ref[...] = qt.reshape(_ED, 1, 8, 128)

    hist_ref[...] += jnp.sum(onehot, axis=0, keepdims=True)
    msum_ref[...] += jnp.sum(mn).reshape(1, 1)

    @pl.when((g == _BG - 1) & (tc == _TC - 1))
    def _fin():
        avg = hist_ref[...] / _N
        ent = jnp.sum(avg * jnp.log(avg + 1e-10))
        perp_ref[...] = jnp.exp(-ent).reshape(1, 1)
        m = msum_ref[...] / (_N * _ED)
        loss_ref[...] = m + _CC * m


def kernel(inputs, W, compute_distances_if_possible):
    del compute_distances_if_possible
    x4 = inputs.reshape(_ED, _BG, 8, _T)
    w2 = jnp.sum(W ** 2, axis=1)[None, :]

    dist, enc, qt, idxo, loss, perp = pl.pallas_call(
        _vq_tile_kernel,
        grid=(_BG, _TC),
        in_specs=[
            pl.BlockSpec((_ED, 1, 8, 128), lambda g, t: (0, g, 0, t)),
            pl.BlockSpec((_NE, _ED), lambda g, t: (0, 0)),
            pl.BlockSpec((1, _NE), lambda g, t: (0, 0)),
        ],
        out_specs=[
            pl.BlockSpec((1, 8, 1, _NE, 128), lambda g, t: (g, 0, t, 0, 0)),
            pl.BlockSpec((1, 8, 1, _NE, 128), lambda g, t: (g, 0, t, 0, 0)),
            pl.BlockSpec((_ED, 1, 8, 128), lambda g, t: (0, g, 0, t)),
            pl.BlockSpec((1, 8, 1, 1, 128), lambda g, t: (g, 0, t, 0, 0)),
            pl.BlockSpec((1, 1), lambda g, t: (0, 0)),
            pl.BlockSpec((1, 1), lambda g, t: (0, 0)),
        ],
        out_shape=[
            jax.ShapeDtypeStruct((_BG, 8, _TC, _NE, 128), jnp.float32),
            jax.ShapeDtypeStruct((_BG, 8, _TC, _NE, 128), jnp.float32),
            jax.ShapeDtypeStruct((_ED, _BG, 8, _T), jnp.float32),
            jax.ShapeDtypeStruct((_BG, 8, _TC, 1, 128), jnp.int32),
            jax.ShapeDtypeStruct((1, 1), jnp.float32),
            jax.ShapeDtypeStruct((1, 1), jnp.float32),
        ],
        scratch_shapes=[
            pltpu.VMEM((1, _NE), jnp.float32),
            pltpu.VMEM((1, 1), jnp.float32),
        ],
    )(x4, W, w2)

    vq_loss = loss[0, 0]
    quantized_out = qt.reshape(_ED, _B, _T)
    perplexity = perp[0, 0]
    encodings_view = enc.reshape(_ED, _NE, 128)
    distances_view = dist.reshape(_ED, _NE, 128)
    return (vq_loss, quantized_out, perplexity, encodings_view,
            distances_view, idxo.reshape(_N, 1))
